# Initial kernel scaffold; baseline (speedup 1.0000x reference)
import jax
import jax.numpy as jnp
from jax.experimental import pallas as pl


def kernel(x):
    raise NotImplementedError("write your pallas kernel here")


# speedup vs baseline: 1.9776x; 1.9776x over previous
import jax
import jax.numpy as jnp
from jax.experimental import pallas as pl


_BLOCK_ROWS = 4096


def _body(x_ref, o_ref):
    i = pl.program_id(0)
    o_ref[...] = x_ref[...] + 1.0

    @pl.when(i == 0)
    def _set_row1():
        o_ref[1, :] = jnp.full((o_ref.shape[1],), 2.0, o_ref.dtype)


def kernel(x):
    m, n = x.shape
    return pl.pallas_call(
        _body,
        grid=(m // _BLOCK_ROWS,),
        in_specs=[pl.BlockSpec((_BLOCK_ROWS, n), lambda i: (i, 0))],
        out_specs=pl.BlockSpec((_BLOCK_ROWS, n), lambda i: (i, 0)),
        out_shape=jax.ShapeDtypeStruct((m, n), x.dtype),
    )(x)
